# Initial kernel scaffold; baseline (speedup 1.0000x reference)
import jax
import jax.numpy as jnp
from jax.experimental import pallas as pl


def kernel(x, edge_index, edge_attr, params):
    raise NotImplementedError("write your pallas kernel here")


# speedup vs baseline: 2.2148x; 2.2148x over previous
import functools

import jax
import jax.numpy as jnp
from jax import lax
from jax.experimental import pallas as pl
from jax.experimental.pallas import tpu as pltpu
from jax.experimental.pallas import tpu_sc as plsc

N = 50000
E = 800000
HID = 64
HALF = 32
LANES = 16
NC, NS = 2, 16
K = 128
NPAD = 50176
E_PAD = 819200
NBLK = E_PAD // K
BPT = NBLK // NS
RPT = NPAD // NS
RB = 896
RBE = 2048

_SC_MESH = plsc.VectorSubcoreMesh(
    core_axis_name="c", subcore_axis_name="s", num_cores=NC, num_subcores=NS)
_SC_PARAMS = pltpu.CompilerParams(use_tc_tiling_on_sc=False)


@functools.partial(
    pl.kernel,
    out_type=jax.ShapeDtypeStruct((NC, NPAD, HALF), jnp.float32),
    mesh=_SC_MESH,
    scratch_types=[
        pltpu.VMEM((K,), jnp.int32),
        pltpu.VMEM((K,), jnp.int32),
        pltpu.VMEM((K, HALF), jnp.float32),
        pltpu.VMEM((K, HALF), jnp.float32),
        pltpu.VMEM((K, HALF), jnp.float32),
        pltpu.VMEM((K, HALF), jnp.float32),
        pltpu.VMEM_SHARED((NPAD, HALF), jnp.float32),
        pltpu.SemaphoreType.DMA,
        pltpu.SemaphoreType.DMA,
    ],
    compiler_params=_SC_PARAMS,
)
def _sc_agg(a_hbm, b_hbm, c_hbm, dst_hbm, src_hbm, zeros_hbm, out_hbm,
            idx_d, idx_s, a_v, b_v, c_v, m_v, acc, sem_a, sem_b):
    cid = lax.axis_index("c")
    sid = lax.axis_index("s")
    row0 = sid * RPT
    pltpu.sync_copy(zeros_hbm.at[cid, pl.ds(row0, RPT)], acc.at[pl.ds(row0, RPT)])
    plsc.subcore_barrier()

    def block(j, carry):
        g = sid * BPT + j
        pltpu.sync_copy(dst_hbm.at[g], idx_d)
        pltpu.sync_copy(src_hbm.at[g], idx_s)
        cp_a = pltpu.async_copy(a_hbm.at[cid].at[idx_d], a_v, sem_a)
        cp_b = pltpu.async_copy(b_hbm.at[cid].at[idx_s], b_v, sem_b)
        pltpu.sync_copy(c_hbm.at[cid, g], c_v)
        cp_a.wait()
        cp_b.wait()

        def rows(r, c2):
            for u in range(2):
                rr = 2 * r + u
                for h in range(HALF // LANES):
                    sl = pl.ds(h * LANES, LANES)
                    m_v[rr, sl] = jnp.maximum(
                        a_v[rr, sl] + b_v[rr, sl] + c_v[rr, sl], 0.0)
            return c2

        lax.fori_loop(0, K // 2, rows, 0, unroll=False)
        pltpu.sync_copy(m_v, acc.at[idx_d], add=True)
        return carry

    lax.fori_loop(0, BPT, block, 0, unroll=False)
    plsc.subcore_barrier()
    pltpu.sync_copy(acc.at[pl.ds(row0, RPT)], out_hbm.at[cid, pl.ds(row0, RPT)])


@functools.partial(
    pl.kernel,
    out_type=jax.ShapeDtypeStruct((NC, NPAD, HALF), jnp.float32),
    mesh=_SC_MESH,
    scratch_types=[
        pltpu.VMEM((K,), jnp.int32),
        pltpu.VMEM((K, HALF), jnp.float32),
        pltpu.VMEM_SHARED((NPAD, HALF), jnp.float32),
    ],
    compiler_params=_SC_PARAMS,
)
def _sc_count(dst_hbm, zeros_hbm, out_hbm, idx_d, ones_v, acc):
    cid = lax.axis_index("c")
    sid = lax.axis_index("s")
    row0 = sid * RPT
    pltpu.sync_copy(zeros_hbm.at[cid, pl.ds(row0, RPT)], acc.at[pl.ds(row0, RPT)])

    def fill(r, c2):
        for h in range(HALF // LANES):
            ones_v[r, pl.ds(h * LANES, LANES)] = jnp.full((LANES,), 1.0, jnp.float32)
        return c2

    lax.fori_loop(0, K, fill, 0, unroll=False)
    plsc.subcore_barrier()

    def block(j, carry):
        g = (cid * NS + sid) * (BPT // NC) + j
        pltpu.sync_copy(dst_hbm.at[g], idx_d)
        pltpu.sync_copy(ones_v, acc.at[idx_d], add=True)
        return carry

    lax.fori_loop(0, BPT // NC, block, 0, unroll=False)
    plsc.subcore_barrier()
    pltpu.sync_copy(acc.at[pl.ds(row0, RPT)], out_hbm.at[cid, pl.ds(row0, RPT)])


def _full(shape):
    return pl.BlockSpec(shape, lambda i: tuple(0 for _ in shape))


def _split_store(ref, full):
    ref[0] = full[:, :HALF]
    ref[1] = full[:, HALF:]


def _tc_encode(xp, e1, eb1, e2, eb2, wd, ws):
    def body(x_ref, e1_ref, eb1_ref, e2_ref, eb2_ref, wd_ref, ws_ref,
             h_ref, a_ref, b_ref):
        h1 = jnp.maximum(jnp.dot(x_ref[...], e1_ref[...],
                                 preferred_element_type=jnp.float32)
                         + eb1_ref[...], 0.0)
        h = jnp.dot(h1, e2_ref[...], preferred_element_type=jnp.float32) + eb2_ref[...]
        h_ref[...] = h
        _split_store(a_ref, jnp.dot(h, wd_ref[...], preferred_element_type=jnp.float32))
        _split_store(b_ref, jnp.dot(h, ws_ref[...], preferred_element_type=jnp.float32))

    return pl.pallas_call(
        body,
        grid=(NPAD // RB,),
        in_specs=[pl.BlockSpec((RB, 8), lambda i: (i, 0)),
                  _full((8, HID)), _full((1, HID)), _full((HID, HID)),
                  _full((1, HID)), _full((HID, HID)), _full((HID, HID))],
        out_specs=[pl.BlockSpec((RB, HID), lambda i: (i, 0)),
                   pl.BlockSpec((NC, RB, HALF), lambda i: (0, i, 0)),
                   pl.BlockSpec((NC, RB, HALF), lambda i: (0, i, 0))],
        out_shape=[jax.ShapeDtypeStruct((NPAD, HID), jnp.float32),
                   jax.ShapeDtypeStruct((NC, NPAD, HALF), jnp.float32),
                   jax.ShapeDtypeStruct((NC, NPAD, HALF), jnp.float32)],
    )(xp, e1, eb1, e2, eb2, wd, ws)


def _tc_edgec(eap, we, b1):
    def body(ea_ref, we_ref, b1_ref, out_ref):
        c = jnp.dot(ea_ref[...], we_ref[...],
                    preferred_element_type=jnp.float32) + b1_ref[...]
        _split_store(out_ref, c)

    return pl.pallas_call(
        body,
        grid=(E_PAD // RBE,),
        in_specs=[pl.BlockSpec((RBE, 8), lambda i: (i, 0)),
                  _full((8, HID)), _full((1, HID))],
        out_specs=pl.BlockSpec((NC, RBE, HALF), lambda i: (0, i, 0)),
        out_shape=jax.ShapeDtypeStruct((NC, E_PAD, HALF), jnp.float32),
    )(eap, we, b1)


def _update_core(h_ref, s_ref, cnt_ref, u1a_ref, g_ref, e2_ref, d_ref,
                 uw2_ref, ub2_ref, lng_ref, lnb_ref):
    h = h_ref[...]
    s = jnp.concatenate([s_ref[0], s_ref[1]], axis=1)
    u_pre = (jnp.dot(h, u1a_ref[...], preferred_element_type=jnp.float32)
             + jnp.dot(s, g_ref[...], preferred_element_type=jnp.float32)
             + cnt_ref[...] * e2_ref[...] + d_ref[...])
    u = jnp.maximum(u_pre, 0.0)
    u2 = jnp.dot(u, uw2_ref[...], preferred_element_type=jnp.float32) + ub2_ref[...]
    r = u2 + h
    mu = jnp.mean(r, axis=-1, keepdims=True)
    var = jnp.mean((r - mu) ** 2, axis=-1, keepdims=True)
    return (r - mu) * jax.lax.rsqrt(var + 1e-5) * lng_ref[...] + lnb_ref[...]


_UPD_SPECS = [pl.BlockSpec((RB, HID), lambda i: (i, 0)),
              pl.BlockSpec((NC, RB, HALF), lambda i: (0, i, 0)),
              pl.BlockSpec((RB, 1), lambda i: (i, 0)),
              _full((HID, HID)), _full((HID, HID)), _full((1, HID)),
              _full((1, HID)), _full((HID, HID)), _full((1, HID)),
              _full((1, HID)), _full((1, HID))]


def _tc_update_prep(h, s, cnt, lw, wd, ws):
    def body(h_ref, s_ref, cnt_ref, u1a_ref, g_ref, e2_ref, d_ref, uw2_ref,
             ub2_ref, lng_ref, lnb_ref, wd_ref, ws_ref, hn_ref, a_ref, b_ref):
        hn = _update_core(h_ref, s_ref, cnt_ref, u1a_ref, g_ref, e2_ref,
                          d_ref, uw2_ref, ub2_ref, lng_ref, lnb_ref)
        hn_ref[...] = hn
        _split_store(a_ref, jnp.dot(hn, wd_ref[...], preferred_element_type=jnp.float32))
        _split_store(b_ref, jnp.dot(hn, ws_ref[...], preferred_element_type=jnp.float32))

    return pl.pallas_call(
        body,
        grid=(NPAD // RB,),
        in_specs=_UPD_SPECS + [_full((HID, HID)), _full((HID, HID))],
        out_specs=[pl.BlockSpec((RB, HID), lambda i: (i, 0)),
                   pl.BlockSpec((NC, RB, HALF), lambda i: (0, i, 0)),
                   pl.BlockSpec((NC, RB, HALF), lambda i: (0, i, 0))],
        out_shape=[jax.ShapeDtypeStruct((NPAD, HID), jnp.float32),
                   jax.ShapeDtypeStruct((NC, NPAD, HALF), jnp.float32),
                   jax.ShapeDtypeStruct((NC, NPAD, HALF), jnp.float32)],
    )(h, s, cnt, *lw, wd, ws)


def _tc_update_decode(h, s, cnt, lw, d1, db1, d2, db2):
    def body(h_ref, s_ref, cnt_ref, u1a_ref, g_ref, e2_ref, d_ref, uw2_ref,
             ub2_ref, lng_ref, lnb_ref, d1_ref, db1_ref, d2_ref, db2_ref,
             out_ref):
        hn = _update_core(h_ref, s_ref, cnt_ref, u1a_ref, g_ref, e2_ref,
                          d_ref, uw2_ref, ub2_ref, lng_ref, lnb_ref)
        o1 = jnp.maximum(jnp.dot(hn, d1_ref[...],
                                 preferred_element_type=jnp.float32)
                         + db1_ref[...], 0.0)
        out_ref[...] = jnp.dot(o1, d2_ref[...],
                               preferred_element_type=jnp.float32) + db2_ref[...]

    return pl.pallas_call(
        body,
        grid=(NPAD // RB,),
        in_specs=_UPD_SPECS + [_full((HID, HID)), _full((1, HID)),
                               _full((HID, 8)), _full((1, 8))],
        out_specs=pl.BlockSpec((RB, 8), lambda i: (i, 0)),
        out_shape=jax.ShapeDtypeStruct((NPAD, 8), jnp.float32),
    )(h, s, cnt, *lw, d1, db1, d2, db2)


def _row(v):
    return v.reshape(1, -1)


def kernel(x, edge_index, edge_attr, params):
    f32 = jnp.float32
    xp = jnp.zeros((NPAD, 8), f32).at[:N, :3].set(x)
    src = edge_index[0]
    dst = edge_index[1]
    dstp = jnp.concatenate([dst, jnp.full((E_PAD - E,), N, jnp.int32)]).reshape(NBLK, K)
    srcp = jnp.concatenate([src, jnp.zeros((E_PAD - E,), jnp.int32)]).reshape(NBLK, K)
    eap = jnp.zeros((E_PAD, 8), f32).at[:E, :3].set(edge_attr)
    zeros2 = jnp.zeros((NC, NPAD, HALF), f32)

    p = params
    cnt2 = _sc_count(dstp, zeros2)
    cnt = (cnt2[0, :, :1] + cnt2[1, :, :1])

    lps = p['layers']
    w1 = lps[0]['msg_w1']
    h, a, b = _tc_encode(xp, jnp.zeros((8, HID), f32).at[:3].set(p['enc_w1']),
                         _row(p['enc_b1']), p['enc_w2'], _row(p['enc_b2']),
                         w1[:HID], w1[HID:2 * HID])

    out = None
    for li, lp in enumerate(lps):
        w1 = lp['msg_w1']
        we = jnp.zeros((8, HID), f32).at[:3].set(w1[2 * HID:])
        c2 = _tc_edgec(eap, we, _row(lp['msg_b1']))
        s = _sc_agg(a, b, c2.reshape(NC, NBLK, K, HALF), dstp, srcp, zeros2)
        u1b = lp['upd_w1'][HID:]
        lw = (lp['upd_w1'][:HID], lp['msg_w2'] @ u1b,
              _row(lp['msg_b2'] @ u1b), _row(lp['upd_b1']),
              lp['upd_w2'], _row(lp['upd_b2']),
              _row(lp['ln_g']), _row(lp['ln_b']))
        if li + 1 < len(lps):
            nw1 = lps[li + 1]['msg_w1']
            h, a, b = _tc_update_prep(h, s, cnt, lw, nw1[:HID], nw1[HID:2 * HID])
        else:
            out = _tc_update_decode(
                h, s, cnt, lw, p['dec_w1'], _row(p['dec_b1']),
                jnp.zeros((HID, 8), f32).at[:, :1].set(p['dec_w2']),
                jnp.zeros((1, 8), f32).at[0, 0].set(p['dec_b2'][0]))
    return out[:N, :1]


# speedup vs baseline: 3.6206x; 1.6348x over previous
import functools

import jax
import jax.numpy as jnp
from jax import lax
from jax.experimental import pallas as pl
from jax.experimental.pallas import tpu as pltpu
from jax.experimental.pallas import tpu_sc as plsc

N = 50000
E = 800000
HID = 64
HALF = 32
LANES = 16
NC, NS = 2, 16
K = 128
NPAD = 50176
E_PAD = 819200
NBLK = E_PAD // K
BPT = NBLK // NS
RPT = NPAD // NS
RB = 896
RBE = 2048

_SC_MESH = plsc.VectorSubcoreMesh(
    core_axis_name="c", subcore_axis_name="s", num_cores=NC, num_subcores=NS)
_SC_PARAMS = pltpu.CompilerParams(use_tc_tiling_on_sc=False)


NB2 = 2
NB4 = 4


@functools.partial(
    pl.kernel,
    out_type=jax.ShapeDtypeStruct((NC, NPAD, HALF), jnp.float32),
    mesh=_SC_MESH,
    scratch_types=(
        [pltpu.VMEM((K,), jnp.int32)] * NB4
        + [pltpu.VMEM((K,), jnp.int32)] * NB4
        + [pltpu.VMEM((K,), jnp.int32)] * NB2
        + [pltpu.VMEM((K, HALF), jnp.float32)] * NB2
        + [pltpu.VMEM((K, HALF), jnp.float32)] * NB2
        + [pltpu.VMEM((K, HALF), jnp.float32)] * NB2
        + [pltpu.VMEM((K * 8 + 8,), jnp.float32)] * NB2
        + [pltpu.VMEM((8, 4 * HALF), jnp.float32)]
        + [pltpu.VMEM_SHARED((NPAD, HALF), jnp.float32)]
        + [pltpu.SemaphoreType.DMA] * (2 * NB4 + 4 * NB2)
    ),
    compiler_params=_SC_PARAMS,
)
def _sc_agg(a_hbm, b_hbm, ea_hbm, we_hbm, dst_hbm, src_hbm, zeros_hbm,
            out_hbm, *refs):
    idx_d = refs[0:NB4]
    idx_s = refs[NB4:2 * NB4]
    idx_c = refs[2 * NB4:2 * NB4 + NB2]
    o = 2 * NB4 + NB2
    a_v = refs[o:o + NB2]
    b_v = refs[o + NB2:o + 2 * NB2]
    m_v = refs[o + 2 * NB2:o + 3 * NB2]
    ea_v = refs[o + 3 * NB2:o + 4 * NB2]
    we_v = refs[o + 4 * NB2]
    acc_ref = refs[o + 4 * NB2 + 1]
    sems = refs[o + 4 * NB2 + 2:]
    s_id_ = sems[0:NB4]
    s_is_ = sems[NB4:2 * NB4]
    s_a = sems[2 * NB4:2 * NB4 + NB2]
    s_b = sems[2 * NB4 + NB2:2 * NB4 + 2 * NB2]
    s_ea = sems[2 * NB4 + 2 * NB2:2 * NB4 + 3 * NB2]
    s_sc = sems[2 * NB4 + 3 * NB2:]
    cid = lax.axis_index("c")
    sid = lax.axis_index("s")
    row0 = sid * RPT
    zcp = pltpu.async_copy(zeros_hbm.at[cid, pl.ds(row0, RPT)],
                           acc_ref.at[pl.ds(row0, RPT)], s_sc[0])
    pltpu.sync_copy(we_hbm, we_v)
    wvec = [we_v[k, pl.ds(cid * HALF + h * LANES, LANES)]
            for k in range(3) for h in range(HALF // LANES)]

    def zfill(r, c2):
        for s in range(NB2):
            for h in range(HALF // LANES):
                m_v[s][r, pl.ds(h * LANES, LANES)] = jnp.zeros((LANES,), jnp.float32)
        return c2

    lax.fori_loop(0, K, zfill, 0)
    for s in range(NB2):
        for i in range(K // LANES):
            idx_c[s][pl.ds(i * LANES, LANES)] = jnp.zeros((LANES,), jnp.int32)
    zcp.wait()
    plsc.subcore_barrier()

    g0 = sid * BPT
    lane_k = [jnp.full((LANES,), k, jnp.int32) for k in range(3)]

    def launch_idx(j, s):
        g = (g0 + j) % NBLK
        pltpu.async_copy(dst_hbm.at[g], idx_d[s], s_id_[s])
        pltpu.async_copy(src_hbm.at[g], idx_s[s], s_is_[s])

    def launch_gather(j, si, s):
        g = (g0 + j) % NBLK
        pltpu.async_copy(a_hbm.at[cid].at[idx_d[si]], a_v[s], s_a[s])
        pltpu.async_copy(b_hbm.at[cid].at[idx_s[si]], b_v[s], s_b[s])
        pltpu.async_copy(ea_hbm.at[g], ea_v[s].at[pl.ds(0, K * 8)], s_ea[s])

    def wait_idx(s):
        pltpu.make_async_copy(dst_hbm.at[0], idx_d[s], s_id_[s]).wait()
        pltpu.make_async_copy(src_hbm.at[0], idx_s[s], s_is_[s]).wait()

    def wait_gather(si, s):
        pltpu.make_async_copy(a_hbm.at[cid].at[idx_d[si]], a_v[s], s_a[s]).wait()
        pltpu.make_async_copy(b_hbm.at[cid].at[idx_s[si]], b_v[s], s_b[s]).wait()
        pltpu.make_async_copy(ea_hbm.at[0], ea_v[s].at[pl.ds(0, K * 8)],
                              s_ea[s]).wait()

    for j in range(3):
        launch_idx(j, j)
    wait_idx(0)
    launch_gather(0, 0, 0)
    for s in range(NB2):
        pltpu.async_copy(m_v[s], acc_ref.at[idx_c[s]], s_sc[s], add=True)

    def block(jj, carry):
        for u in range(NB4):
            j = NB4 * jj + u
            p = u % NB2
            n = (u + 1) % NB2
            i4 = u
            wait_idx((u + 1) % NB4)
            wait_gather(i4, p)
            launch_gather(j + 1, (u + 1) % NB4, n)
            launch_idx(j + 3, (u + 3) % NB4)
            pltpu.make_async_copy(m_v[p], acc_ref.at[idx_c[p]], s_sc[p]).wait()
            for i in range(K // LANES):
                sl = pl.ds(i * LANES, LANES)
                idx_c[p][sl] = idx_d[i4][sl]

            def rows(r, c2):
                v = ea_v[p][pl.ds(8 * r, LANES)]
                eab = [v.at[lane_k[k]].get(mode="promise_in_bounds")
                       for k in range(3)]
                for h in range(HALF // LANES):
                    sl2 = pl.ds(h * LANES, LANES)
                    c = (eab[0] * wvec[h] + eab[1] * wvec[2 + h]
                         + eab[2] * wvec[4 + h])
                    m_v[p][r, sl2] = jnp.maximum(
                        a_v[p][r, sl2] + b_v[p][r, sl2] + c, 0.0)
                return c2

            lax.fori_loop(0, K, rows, 0)
            pltpu.async_copy(m_v[p], acc_ref.at[idx_c[p]], s_sc[p], add=True)
        return carry

    lax.fori_loop(0, BPT // NB4, block, 0)
    wait_gather(0, 0)
    wait_idx(1)
    wait_idx(2)
    for s in range(NB2):
        pltpu.make_async_copy(m_v[s], acc_ref.at[idx_c[s]], s_sc[s]).wait()
    plsc.subcore_barrier()
    pltpu.sync_copy(acc_ref.at[pl.ds(row0, RPT)], out_hbm.at[cid, pl.ds(row0, RPT)])


@functools.partial(
    pl.kernel,
    out_type=jax.ShapeDtypeStruct((NC, NPAD, HALF), jnp.float32),
    mesh=_SC_MESH,
    scratch_types=[
        pltpu.VMEM((K,), jnp.int32),
        pltpu.VMEM((K, HALF), jnp.float32),
        pltpu.VMEM_SHARED((NPAD, HALF), jnp.float32),
    ],
    compiler_params=_SC_PARAMS,
)
def _sc_count(dst_hbm, zeros_hbm, out_hbm, idx_d, ones_v, acc):
    cid = lax.axis_index("c")
    sid = lax.axis_index("s")
    row0 = sid * RPT
    pltpu.sync_copy(zeros_hbm.at[cid, pl.ds(row0, RPT)], acc.at[pl.ds(row0, RPT)])

    def fill(r, c2):
        for h in range(HALF // LANES):
            ones_v[r, pl.ds(h * LANES, LANES)] = jnp.full((LANES,), 1.0, jnp.float32)
        return c2

    lax.fori_loop(0, K, fill, 0, unroll=False)
    plsc.subcore_barrier()

    def block(j, carry):
        g = (cid * NS + sid) * (BPT // NC) + j
        pltpu.sync_copy(dst_hbm.at[g], idx_d)
        pltpu.sync_copy(ones_v, acc.at[idx_d], add=True)
        return carry

    lax.fori_loop(0, BPT // NC, block, 0, unroll=False)
    plsc.subcore_barrier()
    pltpu.sync_copy(acc.at[pl.ds(row0, RPT)], out_hbm.at[cid, pl.ds(row0, RPT)])


def _full(shape):
    return pl.BlockSpec(shape, lambda i: tuple(0 for _ in shape))


def _split_store(ref, full):
    ref[0] = full[:, :HALF]
    ref[1] = full[:, HALF:]


def _tc_encode(xp, e1, eb1, e2, eb2, wd, ws, mb1):
    def body(x_ref, e1_ref, eb1_ref, e2_ref, eb2_ref, wd_ref, ws_ref, mb1_ref,
             h_ref, a_ref, b_ref):
        h1 = jnp.maximum(jnp.dot(x_ref[...], e1_ref[...],
                                 preferred_element_type=jnp.float32)
                         + eb1_ref[...], 0.0)
        h = jnp.dot(h1, e2_ref[...], preferred_element_type=jnp.float32) + eb2_ref[...]
        h_ref[...] = h
        _split_store(a_ref, jnp.dot(h, wd_ref[...],
                                    preferred_element_type=jnp.float32) + mb1_ref[...])
        _split_store(b_ref, jnp.dot(h, ws_ref[...], preferred_element_type=jnp.float32))

    return pl.pallas_call(
        body,
        grid=(NPAD // RB,),
        in_specs=[pl.BlockSpec((RB, 8), lambda i: (i, 0)),
                  _full((8, HID)), _full((1, HID)), _full((HID, HID)),
                  _full((1, HID)), _full((HID, HID)), _full((HID, HID)),
                  _full((1, HID))],
        out_specs=[pl.BlockSpec((RB, HID), lambda i: (i, 0)),
                   pl.BlockSpec((NC, RB, HALF), lambda i: (0, i, 0)),
                   pl.BlockSpec((NC, RB, HALF), lambda i: (0, i, 0))],
        out_shape=[jax.ShapeDtypeStruct((NPAD, HID), jnp.float32),
                   jax.ShapeDtypeStruct((NC, NPAD, HALF), jnp.float32),
                   jax.ShapeDtypeStruct((NC, NPAD, HALF), jnp.float32)],
    )(xp, e1, eb1, e2, eb2, wd, ws, mb1)


def _update_core(h_ref, s_ref, cnt_ref, u1a_ref, g_ref, e2_ref, d_ref,
                 uw2_ref, ub2_ref, lng_ref, lnb_ref):
    h = h_ref[...]
    s = jnp.concatenate([s_ref[0], s_ref[1]], axis=1)
    u_pre = (jnp.dot(h, u1a_ref[...], preferred_element_type=jnp.float32)
             + jnp.dot(s, g_ref[...], preferred_element_type=jnp.float32)
             + cnt_ref[...] * e2_ref[...] + d_ref[...])
    u = jnp.maximum(u_pre, 0.0)
    u2 = jnp.dot(u, uw2_ref[...], preferred_element_type=jnp.float32) + ub2_ref[...]
    r = u2 + h
    mu = jnp.mean(r, axis=-1, keepdims=True)
    var = jnp.mean((r - mu) ** 2, axis=-1, keepdims=True)
    return (r - mu) * jax.lax.rsqrt(var + 1e-5) * lng_ref[...] + lnb_ref[...]


_UPD_SPECS = [pl.BlockSpec((RB, HID), lambda i: (i, 0)),
              pl.BlockSpec((NC, RB, HALF), lambda i: (0, i, 0)),
              pl.BlockSpec((RB, 1), lambda i: (i, 0)),
              _full((HID, HID)), _full((HID, HID)), _full((1, HID)),
              _full((1, HID)), _full((HID, HID)), _full((1, HID)),
              _full((1, HID)), _full((1, HID))]


def _tc_update_prep(h, s, cnt, lw, wd, ws, mb1):
    def body(h_ref, s_ref, cnt_ref, u1a_ref, g_ref, e2_ref, d_ref, uw2_ref,
             ub2_ref, lng_ref, lnb_ref, wd_ref, ws_ref, mb1_ref,
             hn_ref, a_ref, b_ref):
        hn = _update_core(h_ref, s_ref, cnt_ref, u1a_ref, g_ref, e2_ref,
                          d_ref, uw2_ref, ub2_ref, lng_ref, lnb_ref)
        hn_ref[...] = hn
        _split_store(a_ref, jnp.dot(hn, wd_ref[...],
                                    preferred_element_type=jnp.float32) + mb1_ref[...])
        _split_store(b_ref, jnp.dot(hn, ws_ref[...], preferred_element_type=jnp.float32))

    return pl.pallas_call(
        body,
        grid=(NPAD // RB,),
        in_specs=_UPD_SPECS + [_full((HID, HID)), _full((HID, HID)),
                               _full((1, HID))],
        out_specs=[pl.BlockSpec((RB, HID), lambda i: (i, 0)),
                   pl.BlockSpec((NC, RB, HALF), lambda i: (0, i, 0)),
                   pl.BlockSpec((NC, RB, HALF), lambda i: (0, i, 0))],
        out_shape=[jax.ShapeDtypeStruct((NPAD, HID), jnp.float32),
                   jax.ShapeDtypeStruct((NC, NPAD, HALF), jnp.float32),
                   jax.ShapeDtypeStruct((NC, NPAD, HALF), jnp.float32)],
    )(h, s, cnt, *lw, wd, ws, mb1)


def _tc_update_decode(h, s, cnt, lw, d1, db1, d2, db2):
    def body(h_ref, s_ref, cnt_ref, u1a_ref, g_ref, e2_ref, d_ref, uw2_ref,
             ub2_ref, lng_ref, lnb_ref, d1_ref, db1_ref, d2_ref, db2_ref,
             out_ref):
        hn = _update_core(h_ref, s_ref, cnt_ref, u1a_ref, g_ref, e2_ref,
                          d_ref, uw2_ref, ub2_ref, lng_ref, lnb_ref)
        o1 = jnp.maximum(jnp.dot(hn, d1_ref[...],
                                 preferred_element_type=jnp.float32)
                         + db1_ref[...], 0.0)
        out_ref[...] = jnp.dot(o1, d2_ref[...],
                               preferred_element_type=jnp.float32) + db2_ref[...]

    return pl.pallas_call(
        body,
        grid=(NPAD // RB,),
        in_specs=_UPD_SPECS + [_full((HID, HID)), _full((1, HID)),
                               _full((HID, 8)), _full((1, 8))],
        out_specs=pl.BlockSpec((RB, 8), lambda i: (i, 0)),
        out_shape=jax.ShapeDtypeStruct((NPAD, 8), jnp.float32),
    )(h, s, cnt, *lw, d1, db1, d2, db2)


def _row(v):
    return v.reshape(1, -1)


def kernel(x, edge_index, edge_attr, params):
    f32 = jnp.float32
    xp = jnp.zeros((NPAD, 8), f32).at[:N, :3].set(x)
    src = edge_index[0]
    dst = edge_index[1]
    dstp = jnp.concatenate([dst, jnp.full((E_PAD - E,), N, jnp.int32)]).reshape(NBLK, K)
    srcp = jnp.concatenate([src, jnp.zeros((E_PAD - E,), jnp.int32)]).reshape(NBLK, K)
    eap = jnp.zeros((E_PAD, 8), f32).at[:E, :3].set(edge_attr).reshape(NBLK, K * 8)
    zeros2 = jnp.zeros((NC, NPAD, HALF), f32)

    p = params
    cnt2 = _sc_count(dstp, zeros2)
    cnt = (cnt2[0, :, :1] + cnt2[1, :, :1])

    lps = p['layers']
    w1 = lps[0]['msg_w1']
    h, a, b = _tc_encode(xp, jnp.zeros((8, HID), f32).at[:3].set(p['enc_w1']),
                         _row(p['enc_b1']), p['enc_w2'], _row(p['enc_b2']),
                         w1[:HID], w1[HID:2 * HID], _row(lps[0]['msg_b1']))

    out = None
    for li, lp in enumerate(lps):
        w1 = lp['msg_w1']
        we = jnp.zeros((8, 4 * HALF), f32).at[:3, :HID].set(w1[2 * HID:])
        s = _sc_agg(a, b, eap, we, dstp, srcp, zeros2)
        u1b = lp['upd_w1'][HID:]
        lw = (lp['upd_w1'][:HID], lp['msg_w2'] @ u1b,
              _row(lp['msg_b2'] @ u1b), _row(lp['upd_b1']),
              lp['upd_w2'], _row(lp['upd_b2']),
              _row(lp['ln_g']), _row(lp['ln_b']))
        if li + 1 < len(lps):
            nw1 = lps[li + 1]['msg_w1']
            h, a, b = _tc_update_prep(h, s, cnt, lw, nw1[:HID],
                                      nw1[HID:2 * HID],
                                      _row(lps[li + 1]['msg_b1']))
        else:
            out = _tc_update_decode(
                h, s, cnt, lw, p['dec_w1'], _row(p['dec_b1']),
                jnp.zeros((HID, 8), f32).at[:, :1].set(p['dec_w2']),
                jnp.zeros((1, 8), f32).at[0, 0].set(p['dec_b2'][0]))
    return out[:N, :1]
